# baseline (speedup 1.0000x reference)
import functools

import jax
import jax.numpy as jnp
import numpy as np
from jax.experimental import pallas as pl
from jax.experimental.pallas import tpu as pltpu

N = 10000
E = 160000
H = 128
NG = 128
NS = 16
OUT = 128
CUTOFF = 5.0
TMAX = 3200000

TB = 256
TA = 256


def _silu(v):
    return v * jax.nn.sigmoid(v)


def _linear_body(x_ref, w_ref, b_ref, o_ref, act):
    y = jnp.dot(x_ref[...], w_ref[...], preferred_element_type=jnp.float32)
    y = y + b_ref[...]
    if act == "silu":
        y = _silu(y)
    elif act == "relu":
        y = jnp.maximum(y, 0.0)
    o_ref[...] = y


def _linear(x, w, b, act=None, block_rows=2000):
    R, K = x.shape
    F = w.shape[1]
    assert R % block_rows == 0
    grid = (R // block_rows,)
    return pl.pallas_call(
        functools.partial(_linear_body, act=act),
        grid=grid,
        in_specs=[
            pl.BlockSpec((block_rows, K), lambda i: (i, 0)),
            pl.BlockSpec((K, F), lambda i: (0, 0)),
            pl.BlockSpec((1, F), lambda i: (0, 0)),
        ],
        out_specs=pl.BlockSpec((block_rows, F), lambda i: (i, 0)),
        out_shape=jax.ShapeDtypeStruct((R, F), jnp.float32),
    )(x, w, b.reshape(1, F))


def _edge_embed_body(ew_ref, hrow_ref, hcol_ref, wr_ref, br_ref, w1_ref,
                     w2_ref, w3_ref, be_ref, o_ref):
    step = CUTOFF / (NG - 1)
    coeff = -0.5 / step**2
    offset = jax.lax.broadcasted_iota(jnp.int32, (1, NG), 1).astype(jnp.float32) * step
    ew = ew_ref[...]
    rbf = jnp.exp(coeff * (ew - offset) ** 2)
    rbf_h = _silu(jnp.dot(rbf, wr_ref[...], preferred_element_type=jnp.float32)
                  + br_ref[...])
    y = (jnp.dot(hrow_ref[...], w1_ref[...], preferred_element_type=jnp.float32)
         + jnp.dot(hcol_ref[...], w2_ref[...], preferred_element_type=jnp.float32)
         + jnp.dot(rbf_h, w3_ref[...], preferred_element_type=jnp.float32)
         + be_ref[...])
    o_ref[...] = _silu(y)


def _edge_embed(ew, hrow, hcol, params, block_rows=2000):
    grid = (E // block_rows,)
    w1 = params["emb_w"][:H]
    w2 = params["emb_w"][H:2 * H]
    w3 = params["emb_w"][2 * H:]
    return pl.pallas_call(
        _edge_embed_body,
        grid=grid,
        in_specs=[
            pl.BlockSpec((block_rows, 1), lambda i: (i, 0)),
            pl.BlockSpec((block_rows, H), lambda i: (i, 0)),
            pl.BlockSpec((block_rows, H), lambda i: (i, 0)),
            pl.BlockSpec((NG, H), lambda i: (0, 0)),
            pl.BlockSpec((1, H), lambda i: (0, 0)),
            pl.BlockSpec((H, H), lambda i: (0, 0)),
            pl.BlockSpec((H, H), lambda i: (0, 0)),
            pl.BlockSpec((H, H), lambda i: (0, 0)),
            pl.BlockSpec((1, H), lambda i: (0, 0)),
        ],
        out_specs=pl.BlockSpec((block_rows, H), lambda i: (i, 0)),
        out_shape=jax.ShapeDtypeStruct((E, H), jnp.float32),
    )(ew.reshape(E, 1), hrow, hcol, params["rbf_w"],
      params["rbf_b"].reshape(1, H), w1, w2, w3,
      params["emb_b"].reshape(1, H))


def _residual_linear_body(agg_ref, w_ref, b_ref, ea_ref, o_ref):
    y = jnp.dot(agg_ref[...], w_ref[...], preferred_element_type=jnp.float32)
    o_ref[...] = ea_ref[...] + _silu(y + b_ref[...])


def _residual_linear(agg, w, b, ea, block_rows=2000):
    R, K = agg.shape
    F = w.shape[1]
    grid = (R // block_rows,)
    return pl.pallas_call(
        _residual_linear_body,
        grid=grid,
        in_specs=[
            pl.BlockSpec((block_rows, K), lambda i: (i, 0)),
            pl.BlockSpec((K, F), lambda i: (0, 0)),
            pl.BlockSpec((1, F), lambda i: (0, 0)),
            pl.BlockSpec((block_rows, F), lambda i: (i, 0)),
        ],
        out_specs=pl.BlockSpec((block_rows, F), lambda i: (i, 0)),
        out_shape=jax.ShapeDtypeStruct((R, F), jnp.float32),
    )(agg, w, b.reshape(1, F), ea)


def _mlp_head_body(hn_ref, w1_ref, b1_ref, w2_ref, b2_ref, o_ref):
    hn = _silu(hn_ref[...])
    y = jnp.maximum(
        jnp.dot(hn, w1_ref[...], preferred_element_type=jnp.float32)
        + b1_ref[...], 0.0)
    o_ref[...] = (jnp.dot(y, w2_ref[...], preferred_element_type=jnp.float32)
                  + b2_ref[...])


def _mlp_head(hn, params, block_rows=2000):
    grid = (N // block_rows,)
    return pl.pallas_call(
        _mlp_head_body,
        grid=grid,
        in_specs=[
            pl.BlockSpec((block_rows, H), lambda i: (i, 0)),
            pl.BlockSpec((H, H // 2), lambda i: (0, 0)),
            pl.BlockSpec((1, H // 2), lambda i: (0, 0)),
            pl.BlockSpec((H // 2, OUT), lambda i: (0, 0)),
            pl.BlockSpec((1, OUT), lambda i: (0, 0)),
        ],
        out_specs=pl.BlockSpec((block_rows, OUT), lambda i: (i, 0)),
        out_shape=jax.ShapeDtypeStruct((N, OUT), jnp.float32),
    )(hn, params["mlp_w1"], params["mlp_b1"].reshape(1, H // 2),
      params["mlp_w2"], params["mlp_b2"].reshape(1, OUT))


def _triplet_block_body(astart_ref, acnt_ref, qb_ref, nodeb_ref, colb_ref,
                        wsbf_ref, msa_hbm, qat_hbm, nodea_hbm, rowa_hbm,
                        out_ref, ms_scr, qat_scr, na_scr, ra_scr,
                        sem0, sem1, sem2, sem3):
    i = pl.program_id(0)
    astart = astart_ref[i]
    acnt = acnt_ref[i]
    qb = qb_ref[...]
    nodeb = nodeb_ref[...]
    colb = colb_ref[...]
    wsbf = wsbf_ref[...]

    def inner(kk, acc):
        k = astart + kk
        cp0 = pltpu.make_async_copy(
            msa_hbm.at[pl.ds(k * TA, TA), :], ms_scr, sem0)
        cp1 = pltpu.make_async_copy(
            qat_hbm.at[:, pl.ds(k * TA, TA)], qat_scr, sem1)
        cp2 = pltpu.make_async_copy(
            nodea_hbm.at[pl.ds(k, 1), :], na_scr, sem2)
        cp3 = pltpu.make_async_copy(
            rowa_hbm.at[pl.ds(k, 1), :], ra_scr, sem3)
        cp0.start()
        cp1.start()
        cp2.start()
        cp3.start()
        cp0.wait()
        cp1.wait()
        cp2.wait()
        cp3.wait()
        ms = ms_scr[...]
        qat = qat_scr[...]
        na = na_scr[...]
        ra = ra_scr[...]
        hi = jax.lax.Precision.HIGHEST
        c = jnp.dot(qb, qat, preferred_element_type=jnp.float32, precision=hi)
        maskf = ((nodeb == na) & (colb != ra)).astype(jnp.float32)
        p_prev = maskf
        p_cur = c * maskf
        acc = acc + jnp.dot(p_prev, ms, preferred_element_type=jnp.float32,
                            precision=hi) * wsbf[0:1, :]
        acc = acc + jnp.dot(p_cur, ms, preferred_element_type=jnp.float32,
                            precision=hi) * wsbf[1:2, :]
        c2 = c + c
        for s in range(2, NS):
            p_next = c2 * p_cur - p_prev
            acc = acc + jnp.dot(p_next, ms, preferred_element_type=jnp.float32,
                                precision=hi) * wsbf[s:s + 1, :]
            p_prev, p_cur = p_cur, p_next
        return acc

    acc = jax.lax.fori_loop(0, acnt, inner,
                            jnp.zeros((TB, H // 2), jnp.float32))
    out_ref[...] = acc


def _triplet_agg(msa, qat, nodea2, rowa2, qb, nodeb, colb, wsbf,
                 astart, acnt):
    nbt = E // TB
    return pl.pallas_call(
        _triplet_block_body,
        grid=(nbt,),
        in_specs=[
            pl.BlockSpec(memory_space=pltpu.SMEM),
            pl.BlockSpec(memory_space=pltpu.SMEM),
            pl.BlockSpec((TB, 8), lambda i: (i, 0)),
            pl.BlockSpec((TB, 1), lambda i: (i, 0)),
            pl.BlockSpec((TB, 1), lambda i: (i, 0)),
            pl.BlockSpec((NS, H // 2), lambda i: (0, 0)),
            pl.BlockSpec(memory_space=pl.ANY),
            pl.BlockSpec(memory_space=pl.ANY),
            pl.BlockSpec(memory_space=pl.ANY),
            pl.BlockSpec(memory_space=pl.ANY),
        ],
        out_specs=pl.BlockSpec((TB, H // 2), lambda i: (i, 0)),
        out_shape=jax.ShapeDtypeStruct((E, H // 2), jnp.float32),
        scratch_shapes=[
            pltpu.VMEM((TA, H // 2), jnp.float32),
            pltpu.VMEM((8, TA), jnp.float32),
            pltpu.VMEM((1, TA), jnp.int32),
            pltpu.VMEM((1, TA), jnp.int32),
            pltpu.SemaphoreType.DMA,
            pltpu.SemaphoreType.DMA,
            pltpu.SemaphoreType.DMA,
            pltpu.SemaphoreType.DMA,
        ],
    )(astart, acnt, qb, nodeb, colb, wsbf, msa, qat, nodea2, rowa2)


def kernel(x, edge_index, edge_attr, params):
    row = edge_index[0].astype(jnp.int32)
    col = edge_index[1].astype(jnp.int32)
    nbt = E // TB

    col_order = jnp.argsort(col)
    row_order = jnp.argsort(row)
    inv_row = jnp.argsort(row_order)
    counts = jnp.bincount(col, length=N).astype(jnp.int32)
    cum = jnp.concatenate([jnp.zeros((1,), jnp.int32), jnp.cumsum(counts)])

    norm = jnp.sqrt(jnp.sum(edge_attr * edge_attr, axis=1, keepdims=True))
    u = edge_attr / jnp.maximum(norm, 1e-30)
    ua = u[col_order]
    qat = jnp.zeros((8, E), jnp.float32).at[:3, :].set(ua.T)
    qb = jnp.zeros((E, 8), jnp.float32).at[:, :3].set(u[row_order])
    nodea2 = col[col_order].reshape(nbt, TA)
    rowa2 = row[col_order].reshape(nbt, TA)
    rowb = row[row_order]
    nodeb = rowb.reshape(E, 1)
    colb = col[row_order].reshape(E, 1)

    fn = rowb[0::TB]
    ln = rowb[TB - 1::TB]
    astart_elem = cum[fn]
    aend_elem = cum[ln + 1]
    astart = (astart_elem // TA).astype(jnp.int32)
    acnt = jnp.maximum(
        0, (aend_elem + TA - 1) // TA - astart).astype(jnp.int32)

    ew = norm[:, 0]
    h = params["emb"][x]
    ea = _edge_embed(ew, h[row], h[col], params)

    for bl in ("b1", "b2", "b3"):
        m = _linear(ea, params[bl + "_w1"], params[bl + "_b1"], act="silu")
        msa = m[col_order]
        aggb = _triplet_agg(msa, qat, nodea2, rowa2, qb, nodeb, colb,
                            params[bl + "_wsbf"], astart, acnt)
        agg = aggb[inv_row]
        ea = _residual_linear(agg, params[bl + "_w2"], params[bl + "_b2"], ea)

    node_msg = _linear(ea, params["agg2_w"], params["agg2_b"])
    h = h + jax.ops.segment_sum(node_msg, col, num_segments=N)

    m2 = _linear(jnp.concatenate([h[row], h[col]], axis=-1),
                 params["agg3_w1"], params["agg3_b1"], act="relu")
    m2 = _linear(m2, params["agg3_w2"], params["agg3_b2"])
    s = jax.ops.segment_sum(m2, col, num_segments=N)
    cnt = jax.ops.segment_sum(jnp.ones((E,), jnp.float32), col, num_segments=N)
    hn = s / jnp.clip(cnt, 1.0)[:, None]
    out = _mlp_head(hn, params)
    return out, ea


# speedup vs baseline: 1.6932x; 1.6932x over previous
import functools

import jax
import jax.numpy as jnp
import numpy as np
from jax.experimental import pallas as pl
from jax.experimental.pallas import tpu as pltpu

N = 10000
E = 160000
H = 128
NG = 128
NS = 16
OUT = 128
CUTOFF = 5.0
TMAX = 3200000

TB = 256
TA = 256


def _silu(v):
    return v * jax.nn.sigmoid(v)


def _linear_body(x_ref, w_ref, b_ref, o_ref, act):
    y = jnp.dot(x_ref[...], w_ref[...], preferred_element_type=jnp.float32)
    y = y + b_ref[...]
    if act == "silu":
        y = _silu(y)
    elif act == "relu":
        y = jnp.maximum(y, 0.0)
    o_ref[...] = y


def _linear(x, w, b, act=None, block_rows=2000):
    R, K = x.shape
    F = w.shape[1]
    assert R % block_rows == 0
    grid = (R // block_rows,)
    return pl.pallas_call(
        functools.partial(_linear_body, act=act),
        grid=grid,
        in_specs=[
            pl.BlockSpec((block_rows, K), lambda i: (i, 0)),
            pl.BlockSpec((K, F), lambda i: (0, 0)),
            pl.BlockSpec((1, F), lambda i: (0, 0)),
        ],
        out_specs=pl.BlockSpec((block_rows, F), lambda i: (i, 0)),
        out_shape=jax.ShapeDtypeStruct((R, F), jnp.float32),
    )(x, w, b.reshape(1, F))


def _edge_embed_body(ew_ref, hrow_ref, hcol_ref, wr_ref, br_ref, w1_ref,
                     w2_ref, w3_ref, be_ref, o_ref):
    step = CUTOFF / (NG - 1)
    coeff = -0.5 / step**2
    offset = jax.lax.broadcasted_iota(jnp.int32, (1, NG), 1).astype(jnp.float32) * step
    ew = ew_ref[...]
    rbf = jnp.exp(coeff * (ew - offset) ** 2)
    rbf_h = _silu(jnp.dot(rbf, wr_ref[...], preferred_element_type=jnp.float32)
                  + br_ref[...])
    y = (jnp.dot(hrow_ref[...], w1_ref[...], preferred_element_type=jnp.float32)
         + jnp.dot(hcol_ref[...], w2_ref[...], preferred_element_type=jnp.float32)
         + jnp.dot(rbf_h, w3_ref[...], preferred_element_type=jnp.float32)
         + be_ref[...])
    o_ref[...] = _silu(y)


def _edge_embed(ew, hrow, hcol, params, block_rows=2000):
    grid = (E // block_rows,)
    w1 = params["emb_w"][:H]
    w2 = params["emb_w"][H:2 * H]
    w3 = params["emb_w"][2 * H:]
    return pl.pallas_call(
        _edge_embed_body,
        grid=grid,
        in_specs=[
            pl.BlockSpec((block_rows, 1), lambda i: (i, 0)),
            pl.BlockSpec((block_rows, H), lambda i: (i, 0)),
            pl.BlockSpec((block_rows, H), lambda i: (i, 0)),
            pl.BlockSpec((NG, H), lambda i: (0, 0)),
            pl.BlockSpec((1, H), lambda i: (0, 0)),
            pl.BlockSpec((H, H), lambda i: (0, 0)),
            pl.BlockSpec((H, H), lambda i: (0, 0)),
            pl.BlockSpec((H, H), lambda i: (0, 0)),
            pl.BlockSpec((1, H), lambda i: (0, 0)),
        ],
        out_specs=pl.BlockSpec((block_rows, H), lambda i: (i, 0)),
        out_shape=jax.ShapeDtypeStruct((E, H), jnp.float32),
    )(ew.reshape(E, 1), hrow, hcol, params["rbf_w"],
      params["rbf_b"].reshape(1, H), w1, w2, w3,
      params["emb_b"].reshape(1, H))


def _residual_linear_body(agg_ref, w_ref, b_ref, ea_ref, o_ref):
    y = jnp.dot(agg_ref[...], w_ref[...], preferred_element_type=jnp.float32)
    o_ref[...] = ea_ref[...] + _silu(y + b_ref[...])


def _residual_linear(agg, w, b, ea, block_rows=2000):
    R, K = agg.shape
    F = w.shape[1]
    grid = (R // block_rows,)
    return pl.pallas_call(
        _residual_linear_body,
        grid=grid,
        in_specs=[
            pl.BlockSpec((block_rows, K), lambda i: (i, 0)),
            pl.BlockSpec((K, F), lambda i: (0, 0)),
            pl.BlockSpec((1, F), lambda i: (0, 0)),
            pl.BlockSpec((block_rows, F), lambda i: (i, 0)),
        ],
        out_specs=pl.BlockSpec((block_rows, F), lambda i: (i, 0)),
        out_shape=jax.ShapeDtypeStruct((R, F), jnp.float32),
    )(agg, w, b.reshape(1, F), ea)


def _mlp_head_body(hn_ref, w1_ref, b1_ref, w2_ref, b2_ref, o_ref):
    hn = _silu(hn_ref[...])
    y = jnp.maximum(
        jnp.dot(hn, w1_ref[...], preferred_element_type=jnp.float32)
        + b1_ref[...], 0.0)
    o_ref[...] = (jnp.dot(y, w2_ref[...], preferred_element_type=jnp.float32)
                  + b2_ref[...])


def _mlp_head(hn, params, block_rows=2000):
    grid = (N // block_rows,)
    return pl.pallas_call(
        _mlp_head_body,
        grid=grid,
        in_specs=[
            pl.BlockSpec((block_rows, H), lambda i: (i, 0)),
            pl.BlockSpec((H, H // 2), lambda i: (0, 0)),
            pl.BlockSpec((1, H // 2), lambda i: (0, 0)),
            pl.BlockSpec((H // 2, OUT), lambda i: (0, 0)),
            pl.BlockSpec((1, OUT), lambda i: (0, 0)),
        ],
        out_specs=pl.BlockSpec((block_rows, OUT), lambda i: (i, 0)),
        out_shape=jax.ShapeDtypeStruct((N, OUT), jnp.float32),
    )(hn, params["mlp_w1"], params["mlp_b1"].reshape(1, H // 2),
      params["mlp_w2"], params["mlp_b2"].reshape(1, OUT))


def _triplet_block_body(astart_ref, acnt_ref, qb_ref, nodeb_ref, colb_ref,
                        wsbf_ref, msa_hbm, qat_hbm, nodea_hbm, rowa_hbm,
                        out_ref, ms_scr, qat_scr, na_scr, ra_scr,
                        sem0, sem1, sem2, sem3):
    i = pl.program_id(0)
    astart = astart_ref[i]
    acnt = acnt_ref[i]
    qb = qb_ref[...]
    nodeb = nodeb_ref[...]
    colb = colb_ref[...]
    wsbf = wsbf_ref[...]

    def inner(kk, acc):
        k = astart + kk
        cp0 = pltpu.make_async_copy(
            msa_hbm.at[pl.ds(k * TA, TA), :], ms_scr, sem0)
        cp1 = pltpu.make_async_copy(
            qat_hbm.at[:, pl.ds(k * TA, TA)], qat_scr, sem1)
        cp2 = pltpu.make_async_copy(
            nodea_hbm.at[pl.ds(k, 1), :], na_scr, sem2)
        cp3 = pltpu.make_async_copy(
            rowa_hbm.at[pl.ds(k, 1), :], ra_scr, sem3)
        cp0.start()
        cp1.start()
        cp2.start()
        cp3.start()
        cp0.wait()
        cp1.wait()
        cp2.wait()
        cp3.wait()
        ms = ms_scr[...]
        qat = qat_scr[...]
        na = na_scr[...]
        ra = ra_scr[...]
        hi = jax.lax.Precision.DEFAULT
        c = jnp.dot(qb, qat, preferred_element_type=jnp.float32,
                    precision=jax.lax.Precision.HIGHEST)
        maskf = ((nodeb == na) & (colb != ra)).astype(jnp.float32)
        p_prev = maskf
        p_cur = c * maskf
        acc = acc + jnp.dot(p_prev, ms, preferred_element_type=jnp.float32,
                            precision=hi) * wsbf[0:1, :]
        acc = acc + jnp.dot(p_cur, ms, preferred_element_type=jnp.float32,
                            precision=hi) * wsbf[1:2, :]
        c2 = c + c
        for s in range(2, NS):
            p_next = c2 * p_cur - p_prev
            acc = acc + jnp.dot(p_next, ms, preferred_element_type=jnp.float32,
                                precision=hi) * wsbf[s:s + 1, :]
            p_prev, p_cur = p_cur, p_next
        return acc

    acc = jax.lax.fori_loop(0, acnt, inner,
                            jnp.zeros((TB, H // 2), jnp.float32))
    out_ref[...] = acc


def _triplet_agg(msa, qat, nodea2, rowa2, qb, nodeb, colb, wsbf,
                 astart, acnt):
    nbt = E // TB
    return pl.pallas_call(
        _triplet_block_body,
        grid=(nbt,),
        in_specs=[
            pl.BlockSpec(memory_space=pltpu.SMEM),
            pl.BlockSpec(memory_space=pltpu.SMEM),
            pl.BlockSpec((TB, 8), lambda i: (i, 0)),
            pl.BlockSpec((TB, 1), lambda i: (i, 0)),
            pl.BlockSpec((TB, 1), lambda i: (i, 0)),
            pl.BlockSpec((NS, H // 2), lambda i: (0, 0)),
            pl.BlockSpec(memory_space=pl.ANY),
            pl.BlockSpec(memory_space=pl.ANY),
            pl.BlockSpec(memory_space=pl.ANY),
            pl.BlockSpec(memory_space=pl.ANY),
        ],
        out_specs=pl.BlockSpec((TB, H // 2), lambda i: (i, 0)),
        out_shape=jax.ShapeDtypeStruct((E, H // 2), jnp.float32),
        scratch_shapes=[
            pltpu.VMEM((TA, H // 2), jnp.float32),
            pltpu.VMEM((8, TA), jnp.float32),
            pltpu.VMEM((1, TA), jnp.int32),
            pltpu.VMEM((1, TA), jnp.int32),
            pltpu.SemaphoreType.DMA,
            pltpu.SemaphoreType.DMA,
            pltpu.SemaphoreType.DMA,
            pltpu.SemaphoreType.DMA,
        ],
    )(astart, acnt, qb, nodeb, colb, wsbf, msa, qat, nodea2, rowa2)


def kernel(x, edge_index, edge_attr, params):
    row = edge_index[0].astype(jnp.int32)
    col = edge_index[1].astype(jnp.int32)
    nbt = E // TB

    col_order = jnp.argsort(col)
    row_order = jnp.argsort(row)
    inv_row = jnp.argsort(row_order)
    counts = jnp.bincount(col, length=N).astype(jnp.int32)
    cum = jnp.concatenate([jnp.zeros((1,), jnp.int32), jnp.cumsum(counts)])

    norm = jnp.sqrt(jnp.sum(edge_attr * edge_attr, axis=1, keepdims=True))
    u = edge_attr / jnp.maximum(norm, 1e-30)
    ua = u[col_order]
    qat = jnp.zeros((8, E), jnp.float32).at[:3, :].set(ua.T)
    qb = jnp.zeros((E, 8), jnp.float32).at[:, :3].set(u[row_order])
    nodea2 = col[col_order].reshape(nbt, TA)
    rowa2 = row[col_order].reshape(nbt, TA)
    rowb = row[row_order]
    nodeb = rowb.reshape(E, 1)
    colb = col[row_order].reshape(E, 1)

    fn = rowb[0::TB]
    ln = rowb[TB - 1::TB]
    astart_elem = cum[fn]
    aend_elem = cum[ln + 1]
    astart = (astart_elem // TA).astype(jnp.int32)
    acnt = jnp.maximum(
        0, (aend_elem + TA - 1) // TA - astart).astype(jnp.int32)

    ew = norm[:, 0]
    h = params["emb"][x]
    ea = _edge_embed(ew, h[row], h[col], params)

    for bl in ("b1", "b2", "b3"):
        m = _linear(ea, params[bl + "_w1"], params[bl + "_b1"], act="silu")
        msa = m[col_order]
        aggb = _triplet_agg(msa, qat, nodea2, rowa2, qb, nodeb, colb,
                            params[bl + "_wsbf"], astart, acnt)
        agg = aggb[inv_row]
        ea = _residual_linear(agg, params[bl + "_w2"], params[bl + "_b2"], ea)

    node_msg = _linear(ea, params["agg2_w"], params["agg2_b"])
    h = h + jax.ops.segment_sum(node_msg, col, num_segments=N)

    m2 = _linear(jnp.concatenate([h[row], h[col]], axis=-1),
                 params["agg3_w1"], params["agg3_b1"], act="relu")
    m2 = _linear(m2, params["agg3_w2"], params["agg3_b2"])
    s = jax.ops.segment_sum(m2, col, num_segments=N)
    cnt = jax.ops.segment_sum(jnp.ones((E,), jnp.float32), col, num_segments=N)
    hn = s / jnp.clip(cnt, 1.0)[:, None]
    out = _mlp_head(hn, params)
    return out, ea
